# Initial kernel scaffold; baseline (speedup 1.0000x reference)
import jax
import jax.numpy as jnp
from jax.experimental import pallas as pl


def kernel(x, edge_index, fc1_w, fc1_b, gcn_w0, gcn_b0, gcn_w1, gcn_b1, gcn_w2, gcn_b2, lin_w0, lin_b0, lin_w1, lin_b1, lin_w2, lin_b2, fc2_w, fc2_b, sv_w, sv_b):
    raise NotImplementedError("write your pallas kernel here")


# speedup vs baseline: 3.5742x; 3.5742x over previous
import functools

import jax
import jax.numpy as jnp
from jax import lax
from jax.experimental import pallas as pl
from jax.experimental.pallas import tpu as pltpu, tpu_sc as plsc

N = 10000
E = 160000
FEAT = 256
H = 512
NCHUNK = 4
CW = 128
EB = 128
NBLK = E // EB
NS = 16
RPT = 624
RPT_LAST = N - 15 * RPT
BN = 400
GRID = N // BN

_mesh = plsc.VectorSubcoreMesh(core_axis_name="c", subcore_axis_name="s")


def _copy_tile_rows(sub, src_at, dst_at):
    @pl.when(sub < 15)
    def _():
        pltpu.sync_copy(src_at(RPT), dst_at(RPT))

    @pl.when(sub == 15)
    def _():
        pltpu.sync_copy(src_at(RPT_LAST), dst_at(RPT_LAST))


def _zero_vmem_2d(ref, nrows, ncols):
    ngrp = ncols // 16

    def zrow(r, _):
        def zcol(q, _):
            ref[r, pl.ds(q * 16, 16)] = jnp.zeros((16,), jnp.float32)
            return 0

        lax.fori_loop(0, ngrp, zcol, 0)
        return 0

    lax.fori_loop(0, nrows, zrow, 0)


@functools.partial(
    pl.kernel,
    mesh=_mesh,
    out_type=jax.ShapeDtypeStruct((2, N, CW), jnp.float32),
    scratch_types=[
        pltpu.VMEM((EB,), jnp.int32),
        pltpu.VMEM((EB, CW), jnp.float32),
        pltpu.VMEM((16, CW), jnp.float32),
        pltpu.VMEM_SHARED((N, CW), jnp.float32),
    ],
)
def _deg_kernel(ei_hbm, out_hbm, idx_v, ones_v, zbuf_v, acc_sh):
    core = lax.axis_index("c")
    sub = lax.axis_index("s")
    start = sub * RPT

    def fill_ones(r, _):
        def fcol(q, _):
            ones_v[r, pl.ds(q * 16, 16)] = jnp.ones((16,), jnp.float32)
            return 0

        lax.fori_loop(0, CW // 16, fcol, 0)
        return 0

    lax.fori_loop(0, EB, fill_ones, 0)
    _zero_vmem_2d(zbuf_v, 16, CW)
    nzero = (RPT // 16) + jnp.where(sub == 15, 1, 0)

    for c in range(2):
        @pl.when(core == c)
        def _(c=c):
            def zbody(k, _):
                pltpu.sync_copy(zbuf_v, acc_sh.at[pl.ds(start + k * 16, 16)])
                return 0

            lax.fori_loop(0, nzero, zbody, 0)
            plsc.subcore_barrier()
            nblk = 78 + jnp.where(sub < 2, 1, 0)

            def body(j, _):
                g = sub + NS * j
                pltpu.sync_copy(ei_hbm.at[c, pl.ds(g * EB, EB)], idx_v)
                pltpu.sync_copy(ones_v, acc_sh.at[idx_v], add=True)
                return 0

            lax.fori_loop(0, nblk, body, 0)
            plsc.subcore_barrier()
            _copy_tile_rows(sub,
                            lambda n: acc_sh.at[pl.ds(start, n)],
                            lambda n: out_hbm.at[c, pl.ds(start, n)])


@functools.partial(
    pl.kernel,
    mesh=_mesh,
    out_type=jax.ShapeDtypeStruct((NCHUNK, N, CW), jnp.float32),
    scratch_types=[
        pltpu.VMEM((EB,), jnp.int32),
        pltpu.VMEM((EB,), jnp.int32),
        pltpu.VMEM((EB, CW), jnp.float32),
        pltpu.VMEM((16, CW), jnp.float32),
        pltpu.VMEM_SHARED((N, CW), jnp.float32),
        pltpu.SemaphoreType.DMA,
    ],
)
def _agg_kernel(hn0_hbm, hn1_hbm, hn2_hbm, hn3_hbm, ei_hbm, out_hbm,
                src_v, dst_v, rows_v, zbuf_v, acc_sh, sem):
    hn_chunks = (hn0_hbm, hn1_hbm, hn2_hbm, hn3_hbm)
    core = lax.axis_index("c")
    sub = lax.axis_index("s")
    start = sub * RPT
    _zero_vmem_2d(zbuf_v, 16, CW)
    nzero = (RPT // 16) + jnp.where(sub == 15, 1, 0)

    for c in range(NCHUNK):
        @pl.when(core == (c % 2))
        def _(c=c):
            def zbody(k, _):
                pltpu.sync_copy(zbuf_v, acc_sh.at[pl.ds(start + k * 16, 16)])
                return 0

            lax.fori_loop(0, nzero, zbody, 0)
            plsc.subcore_barrier()
            nblk = 78 + jnp.where(sub < 2, 1, 0)

            def body(j, _):
                g = sub + NS * j
                pltpu.sync_copy(ei_hbm.at[0, pl.ds(g * EB, EB)], src_v)
                pltpu.sync_copy(ei_hbm.at[1, pl.ds(g * EB, EB)], dst_v)
                pltpu.async_copy(hn_chunks[c].at[src_v], rows_v, sem).wait()
                pltpu.sync_copy(rows_v, acc_sh.at[dst_v], add=True)
                return 0

            lax.fori_loop(0, nblk, body, 0)
            plsc.subcore_barrier()
            _copy_tile_rows(sub,
                            lambda n: acc_sh.at[pl.ds(start, n)],
                            lambda n: out_hbm.at[c, pl.ds(start, n)])


def _inv_sqrt_deg(deg16):
    return lax.rsqrt(jnp.maximum(deg16[:, 0:1], 1.0))


def _fc1_body(x_ref, w_ref, b_ref, dego_ref, z_ref, hn0_ref, hn1_ref,
              hn2_ref, hn3_ref, ss_ref):
    i = pl.program_id(0)
    z = jnp.dot(x_ref[...], w_ref[...], preferred_element_type=jnp.float32)
    z = z + b_ref[...]
    z_ref[...] = z
    hn = z * _inv_sqrt_deg(dego_ref[...])
    for c, hr in enumerate((hn0_ref, hn1_ref, hn2_ref, hn3_ref)):
        hr[...] = hn[:, c * CW:(c + 1) * CW]
    ssq = jnp.sum(z * z, axis=0, keepdims=True)

    @pl.when(i == 0)
    def _():
        ss_ref[...] = ssq

    @pl.when(i > 0)
    def _():
        ss_ref[...] = ss_ref[...] + ssq


_fc1_call = pl.pallas_call(
    _fc1_body,
    grid=(GRID,),
    in_specs=[
        pl.BlockSpec((BN, FEAT), lambda i: (i, 0)),
        pl.BlockSpec((FEAT, H), lambda i: (0, 0)),
        pl.BlockSpec((1, H), lambda i: (0, 0)),
        pl.BlockSpec((BN, CW), lambda i: (i, 0)),
    ],
    out_specs=[
        pl.BlockSpec((BN, H), lambda i: (i, 0)),
        pl.BlockSpec((BN, CW), lambda i: (i, 0)),
        pl.BlockSpec((BN, CW), lambda i: (i, 0)),
        pl.BlockSpec((BN, CW), lambda i: (i, 0)),
        pl.BlockSpec((BN, CW), lambda i: (i, 0)),
        pl.BlockSpec((1, H), lambda i: (0, 0)),
    ],
    out_shape=[
        jax.ShapeDtypeStruct((N, H), jnp.float32),
        jax.ShapeDtypeStruct((N, CW), jnp.float32),
        jax.ShapeDtypeStruct((N, CW), jnp.float32),
        jax.ShapeDtypeStruct((N, CW), jnp.float32),
        jax.ShapeDtypeStruct((N, CW), jnp.float32),
        jax.ShapeDtypeStruct((1, H), jnp.float32),
    ],
)


def _layer_body(z_ref, ssp_ref, agg_ref, degi_ref, dego_ref,
                wg_ref, bg_ref, wl_ref, bl_ref,
                zo_ref, hn0_ref, hn1_ref, hn2_ref, hn3_ref, ss_ref, *, relu):
    i = pl.program_id(0)
    inv_cn = 1.0 / jnp.maximum(jnp.sqrt(ssp_ref[...]), 1e-12)
    h = z_ref[...] * inv_cn
    agg = jnp.concatenate(
        [agg_ref[0], agg_ref[1], agg_ref[2], agg_ref[3]], axis=1)
    aggs = agg * inv_cn * _inv_sqrt_deg(degi_ref[...])
    pre = jnp.dot(aggs, wg_ref[...], preferred_element_type=jnp.float32)
    pre = pre + bg_ref[...]
    pre = pre + jnp.dot(h, wl_ref[...], preferred_element_type=jnp.float32)
    pre = pre + bl_ref[...]
    z = jnp.maximum(pre, 0.0) if relu else pre
    zo_ref[...] = z
    hn = z * _inv_sqrt_deg(dego_ref[...])
    for c, hr in enumerate((hn0_ref, hn1_ref, hn2_ref, hn3_ref)):
        hr[...] = hn[:, c * CW:(c + 1) * CW]
    ssq = jnp.sum(z * z, axis=0, keepdims=True)

    @pl.when(i == 0)
    def _():
        ss_ref[...] = ssq

    @pl.when(i > 0)
    def _():
        ss_ref[...] = ss_ref[...] + ssq


def _make_layer_call(relu):
    return pl.pallas_call(
        functools.partial(_layer_body, relu=relu),
        grid=(GRID,),
        in_specs=[
            pl.BlockSpec((BN, H), lambda i: (i, 0)),
            pl.BlockSpec((1, H), lambda i: (0, 0)),
            pl.BlockSpec((NCHUNK, BN, CW), lambda i: (0, i, 0)),
            pl.BlockSpec((BN, CW), lambda i: (i, 0)),
            pl.BlockSpec((BN, CW), lambda i: (i, 0)),
            pl.BlockSpec((H, H), lambda i: (0, 0)),
            pl.BlockSpec((1, H), lambda i: (0, 0)),
            pl.BlockSpec((H, H), lambda i: (0, 0)),
            pl.BlockSpec((1, H), lambda i: (0, 0)),
        ],
        out_specs=[
            pl.BlockSpec((BN, H), lambda i: (i, 0)),
            pl.BlockSpec((BN, CW), lambda i: (i, 0)),
            pl.BlockSpec((BN, CW), lambda i: (i, 0)),
            pl.BlockSpec((BN, CW), lambda i: (i, 0)),
            pl.BlockSpec((BN, CW), lambda i: (i, 0)),
            pl.BlockSpec((1, H), lambda i: (0, 0)),
        ],
        out_shape=[
            jax.ShapeDtypeStruct((N, H), jnp.float32),
            jax.ShapeDtypeStruct((N, CW), jnp.float32),
            jax.ShapeDtypeStruct((N, CW), jnp.float32),
            jax.ShapeDtypeStruct((N, CW), jnp.float32),
            jax.ShapeDtypeStruct((N, CW), jnp.float32),
            jax.ShapeDtypeStruct((1, H), jnp.float32),
        ],
    )


_layer_relu = _make_layer_call(True)
_layer_last = _make_layer_call(False)


def _head_body(row_ref, ss_ref, w2_ref, b2_ref, sv_ref, svb_ref, out_ref):
    r = row_ref[...] / jnp.maximum(jnp.sqrt(ss_ref[...]), 1e-12)
    h3 = jnp.dot(r, w2_ref[...], preferred_element_type=jnp.float32)
    h3 = jnp.maximum(h3 + b2_ref[...], 0.0)
    out_ref[...] = jnp.dot(h3, sv_ref[...],
                           preferred_element_type=jnp.float32) + svb_ref[...]


_head_call = pl.pallas_call(
    _head_body,
    out_shape=jax.ShapeDtypeStruct((1, 1), jnp.float32),
)


def kernel(x, edge_index, fc1_w, fc1_b, gcn_w0, gcn_b0, gcn_w1, gcn_b1,
           gcn_w2, gcn_b2, lin_w0, lin_b0, lin_w1, lin_b1, lin_w2, lin_b2,
           fc2_w, fc2_b, sv_w, sv_b):
    deg16 = _deg_kernel(edge_index)
    dego16 = deg16[0]
    degi16 = deg16[1]

    z0, h0a, h0b, h0c, h0d, ss0 = _fc1_call(x, fc1_w, fc1_b.reshape(1, H),
                                            dego16)
    agg0 = _agg_kernel(h0a, h0b, h0c, h0d, edge_index)
    z1, h1a, h1b, h1c, h1d, ss1 = _layer_relu(
        z0, ss0, agg0, degi16, dego16,
        gcn_w0, gcn_b0.reshape(1, H), lin_w0, lin_b0.reshape(1, H))
    agg1 = _agg_kernel(h1a, h1b, h1c, h1d, edge_index)
    z2, h2a, h2b, h2c, h2d, ss2 = _layer_relu(
        z1, ss1, agg1, degi16, dego16,
        gcn_w1, gcn_b1.reshape(1, H), lin_w1, lin_b1.reshape(1, H))
    agg2 = _agg_kernel(h2a, h2b, h2c, h2d, edge_index)
    z3, _h3a, _h3b, _h3c, _h3d, ss3 = _layer_last(
        z2, ss2, agg2, degi16, dego16,
        gcn_w2, gcn_b2.reshape(1, H), lin_w2, lin_b2.reshape(1, H))
    value = _head_call(z3[N - 1:N], ss3, fc2_w, fc2_b.reshape(1, H),
                       sv_w, sv_b.reshape(1, 1))
    return value.reshape(1)


# speedup vs baseline: 4.7553x; 1.3305x over previous
import functools

import jax
import jax.numpy as jnp
from jax import lax
from jax.experimental import pallas as pl
from jax.experimental.pallas import tpu as pltpu, tpu_sc as plsc

N = 10000
E = 160000
FEAT = 256
H = 512
NCHUNK = 4
CW = 128
EB = 128
NBLK = E // EB
NS = 16
RPT = 624
RPT_LAST = N - 15 * RPT
BN = 400
GRID = N // BN

_mesh = plsc.VectorSubcoreMesh(core_axis_name="c", subcore_axis_name="s")


def _copy_tile_rows(sub, src_at, dst_at):
    @pl.when(sub < 15)
    def _():
        pltpu.sync_copy(src_at(RPT), dst_at(RPT))

    @pl.when(sub == 15)
    def _():
        pltpu.sync_copy(src_at(RPT_LAST), dst_at(RPT_LAST))


def _zero_vmem_2d(ref, nrows, ncols):
    ngrp = ncols // 16

    def zrow(r, _):
        def zcol(q, _):
            ref[r, pl.ds(q * 16, 16)] = jnp.zeros((16,), jnp.float32)
            return 0

        lax.fori_loop(0, ngrp, zcol, 0)
        return 0

    lax.fori_loop(0, nrows, zrow, 0)


@functools.partial(
    pl.kernel,
    mesh=_mesh,
    out_type=jax.ShapeDtypeStruct((2, N, CW), jnp.float32),
    scratch_types=[
        pltpu.VMEM((EB,), jnp.int32),
        pltpu.VMEM((EB, CW), jnp.float32),
        pltpu.VMEM((16, CW), jnp.float32),
        pltpu.VMEM_SHARED((N, CW), jnp.float32),
    ],
)
def _deg_kernel(ei_hbm, out_hbm, idx_v, ones_v, zbuf_v, acc_sh):
    core = lax.axis_index("c")
    sub = lax.axis_index("s")
    start = sub * RPT

    def fill_ones(r, _):
        def fcol(q, _):
            ones_v[r, pl.ds(q * 16, 16)] = jnp.ones((16,), jnp.float32)
            return 0

        lax.fori_loop(0, CW // 16, fcol, 0)
        return 0

    lax.fori_loop(0, EB, fill_ones, 0)
    _zero_vmem_2d(zbuf_v, 16, CW)
    nzero = (RPT // 16) + jnp.where(sub == 15, 1, 0)

    for c in range(2):
        @pl.when(core == c)
        def _(c=c):
            def zbody(k, _):
                pltpu.sync_copy(zbuf_v, acc_sh.at[pl.ds(start + k * 16, 16)])
                return 0

            lax.fori_loop(0, nzero, zbody, 0)
            plsc.subcore_barrier()
            nblk = 78 + jnp.where(sub < 2, 1, 0)

            def body(j, _):
                g = sub + NS * j
                pltpu.sync_copy(ei_hbm.at[c, pl.ds(g * EB, EB)], idx_v)
                pltpu.sync_copy(ones_v, acc_sh.at[idx_v], add=True)
                return 0

            lax.fori_loop(0, nblk, body, 0)
            plsc.subcore_barrier()
            _copy_tile_rows(sub,
                            lambda n: acc_sh.at[pl.ds(start, n)],
                            lambda n: out_hbm.at[c, pl.ds(start, n)])


@functools.partial(
    pl.kernel,
    mesh=_mesh,
    out_type=jax.ShapeDtypeStruct((NCHUNK, N, CW), jnp.float32),
    scratch_types=[
        pltpu.VMEM((EB,), jnp.int32),
        pltpu.VMEM((EB,), jnp.int32),
        pltpu.VMEM((EB,), jnp.int32),
        pltpu.VMEM((EB,), jnp.int32),
        pltpu.VMEM((EB, CW), jnp.float32),
        pltpu.VMEM((EB, CW), jnp.float32),
        pltpu.VMEM((104, CW), jnp.float32),
        pltpu.VMEM_SHARED((N, CW), jnp.float32),
        pltpu.SemaphoreType.DMA,
        pltpu.SemaphoreType.DMA,
        pltpu.SemaphoreType.DMA,
    ],
)
def _agg_kernel(hn0_hbm, hn1_hbm, hn2_hbm, hn3_hbm, ei_hbm, out_hbm,
                src0_v, dst0_v, src1_v, dst1_v, rows0_v, rows1_v,
                zbuf_v, acc_sh, gsem0, gsem1, ssem):
    hn_chunks = (hn0_hbm, hn1_hbm, hn2_hbm, hn3_hbm)
    core = lax.axis_index("c")
    sub = lax.axis_index("s")
    start = sub * RPT
    _zero_vmem_2d(zbuf_v, 104, CW)

    for c in range(NCHUNK):
        @pl.when(core == (c % 2))
        def _(c=c):
            def zbody(k, _):
                pltpu.sync_copy(zbuf_v, acc_sh.at[pl.ds(start + k * 104, 104)])
                return 0

            lax.fori_loop(0, RPT // 104, zbody, 0)

            @pl.when(sub == 15)
            def _():
                pltpu.sync_copy(zbuf_v.at[pl.ds(0, 16)],
                                acc_sh.at[pl.ds(start + RPT, 16)])

            plsc.subcore_barrier()

            def pair(p, _):
                g0 = sub + NS * (2 * p)
                g1 = g0 + NS
                pltpu.sync_copy(ei_hbm.at[0, pl.ds(g0 * EB, EB)], src0_v)
                pltpu.sync_copy(ei_hbm.at[1, pl.ds(g0 * EB, EB)], dst0_v)
                cp0 = pltpu.async_copy(hn_chunks[c].at[src0_v], rows0_v, gsem0)
                pltpu.sync_copy(ei_hbm.at[0, pl.ds(g1 * EB, EB)], src1_v)
                pltpu.sync_copy(ei_hbm.at[1, pl.ds(g1 * EB, EB)], dst1_v)
                cp1 = pltpu.async_copy(hn_chunks[c].at[src1_v], rows1_v, gsem1)
                cp0.wait()
                pltpu.sync_copy(rows0_v, acc_sh.at[dst0_v], add=True)
                cp1.wait()
                pltpu.sync_copy(rows1_v, acc_sh.at[dst1_v], add=True)
                return 0

            lax.fori_loop(0, 39, pair, 0)

            @pl.when(sub < 2)
            def _():
                g = sub + NS * 78
                pltpu.sync_copy(ei_hbm.at[0, pl.ds(g * EB, EB)], src0_v)
                pltpu.sync_copy(ei_hbm.at[1, pl.ds(g * EB, EB)], dst0_v)
                pltpu.async_copy(hn_chunks[c].at[src0_v], rows0_v,
                                 gsem0).wait()
                pltpu.sync_copy(rows0_v, acc_sh.at[dst0_v], add=True)

            plsc.subcore_barrier()
            _copy_tile_rows(sub,
                            lambda n: acc_sh.at[pl.ds(start, n)],
                            lambda n: out_hbm.at[c, pl.ds(start, n)])


def _inv_sqrt_deg(deg16):
    return lax.rsqrt(jnp.maximum(deg16[:, 0:1], 1.0))


def _fc1_body(x_ref, w_ref, b_ref, dego_ref, z_ref, hn0_ref, hn1_ref,
              hn2_ref, hn3_ref, ss_ref):
    i = pl.program_id(0)
    z = jnp.dot(x_ref[...], w_ref[...], preferred_element_type=jnp.float32)
    z = z + b_ref[...]
    z_ref[...] = z
    hn = z * _inv_sqrt_deg(dego_ref[...])
    for c, hr in enumerate((hn0_ref, hn1_ref, hn2_ref, hn3_ref)):
        hr[...] = hn[:, c * CW:(c + 1) * CW]
    ssq = jnp.sum(z * z, axis=0, keepdims=True)

    @pl.when(i == 0)
    def _():
        ss_ref[...] = ssq

    @pl.when(i > 0)
    def _():
        ss_ref[...] = ss_ref[...] + ssq


_fc1_call = pl.pallas_call(
    _fc1_body,
    grid=(GRID,),
    in_specs=[
        pl.BlockSpec((BN, FEAT), lambda i: (i, 0)),
        pl.BlockSpec((FEAT, H), lambda i: (0, 0)),
        pl.BlockSpec((1, H), lambda i: (0, 0)),
        pl.BlockSpec((BN, CW), lambda i: (i, 0)),
    ],
    out_specs=[
        pl.BlockSpec((BN, H), lambda i: (i, 0)),
        pl.BlockSpec((BN, CW), lambda i: (i, 0)),
        pl.BlockSpec((BN, CW), lambda i: (i, 0)),
        pl.BlockSpec((BN, CW), lambda i: (i, 0)),
        pl.BlockSpec((BN, CW), lambda i: (i, 0)),
        pl.BlockSpec((1, H), lambda i: (0, 0)),
    ],
    out_shape=[
        jax.ShapeDtypeStruct((N, H), jnp.float32),
        jax.ShapeDtypeStruct((N, CW), jnp.float32),
        jax.ShapeDtypeStruct((N, CW), jnp.float32),
        jax.ShapeDtypeStruct((N, CW), jnp.float32),
        jax.ShapeDtypeStruct((N, CW), jnp.float32),
        jax.ShapeDtypeStruct((1, H), jnp.float32),
    ],
)


def _layer_body(z_ref, ssp_ref, agg_ref, degi_ref, dego_ref,
                wg_ref, bg_ref, wl_ref, bl_ref,
                zo_ref, hn0_ref, hn1_ref, hn2_ref, hn3_ref, ss_ref, *, relu):
    i = pl.program_id(0)
    inv_cn = 1.0 / jnp.maximum(jnp.sqrt(ssp_ref[...]), 1e-12)
    h = z_ref[...] * inv_cn
    agg = jnp.concatenate(
        [agg_ref[0], agg_ref[1], agg_ref[2], agg_ref[3]], axis=1)
    aggs = agg * inv_cn * _inv_sqrt_deg(degi_ref[...])
    pre = jnp.dot(aggs, wg_ref[...], preferred_element_type=jnp.float32)
    pre = pre + bg_ref[...]
    pre = pre + jnp.dot(h, wl_ref[...], preferred_element_type=jnp.float32)
    pre = pre + bl_ref[...]
    z = jnp.maximum(pre, 0.0) if relu else pre
    zo_ref[...] = z
    hn = z * _inv_sqrt_deg(dego_ref[...])
    for c, hr in enumerate((hn0_ref, hn1_ref, hn2_ref, hn3_ref)):
        hr[...] = hn[:, c * CW:(c + 1) * CW]
    ssq = jnp.sum(z * z, axis=0, keepdims=True)

    @pl.when(i == 0)
    def _():
        ss_ref[...] = ssq

    @pl.when(i > 0)
    def _():
        ss_ref[...] = ss_ref[...] + ssq


def _make_layer_call(relu):
    return pl.pallas_call(
        functools.partial(_layer_body, relu=relu),
        grid=(GRID,),
        in_specs=[
            pl.BlockSpec((BN, H), lambda i: (i, 0)),
            pl.BlockSpec((1, H), lambda i: (0, 0)),
            pl.BlockSpec((NCHUNK, BN, CW), lambda i: (0, i, 0)),
            pl.BlockSpec((BN, CW), lambda i: (i, 0)),
            pl.BlockSpec((BN, CW), lambda i: (i, 0)),
            pl.BlockSpec((H, H), lambda i: (0, 0)),
            pl.BlockSpec((1, H), lambda i: (0, 0)),
            pl.BlockSpec((H, H), lambda i: (0, 0)),
            pl.BlockSpec((1, H), lambda i: (0, 0)),
        ],
        out_specs=[
            pl.BlockSpec((BN, H), lambda i: (i, 0)),
            pl.BlockSpec((BN, CW), lambda i: (i, 0)),
            pl.BlockSpec((BN, CW), lambda i: (i, 0)),
            pl.BlockSpec((BN, CW), lambda i: (i, 0)),
            pl.BlockSpec((BN, CW), lambda i: (i, 0)),
            pl.BlockSpec((1, H), lambda i: (0, 0)),
        ],
        out_shape=[
            jax.ShapeDtypeStruct((N, H), jnp.float32),
            jax.ShapeDtypeStruct((N, CW), jnp.float32),
            jax.ShapeDtypeStruct((N, CW), jnp.float32),
            jax.ShapeDtypeStruct((N, CW), jnp.float32),
            jax.ShapeDtypeStruct((N, CW), jnp.float32),
            jax.ShapeDtypeStruct((1, H), jnp.float32),
        ],
    )


_layer_relu = _make_layer_call(True)
_layer_last = _make_layer_call(False)


def _head_body(row_ref, ss_ref, w2_ref, b2_ref, sv_ref, svb_ref, out_ref):
    r = row_ref[...] / jnp.maximum(jnp.sqrt(ss_ref[...]), 1e-12)
    h3 = jnp.dot(r, w2_ref[...], preferred_element_type=jnp.float32)
    h3 = jnp.maximum(h3 + b2_ref[...], 0.0)
    out_ref[...] = jnp.dot(h3, sv_ref[...],
                           preferred_element_type=jnp.float32) + svb_ref[...]


_head_call = pl.pallas_call(
    _head_body,
    out_shape=jax.ShapeDtypeStruct((1, 1), jnp.float32),
)


def kernel(x, edge_index, fc1_w, fc1_b, gcn_w0, gcn_b0, gcn_w1, gcn_b1,
           gcn_w2, gcn_b2, lin_w0, lin_b0, lin_w1, lin_b1, lin_w2, lin_b2,
           fc2_w, fc2_b, sv_w, sv_b):
    deg16 = _deg_kernel(edge_index)
    dego16 = deg16[0]
    degi16 = deg16[1]

    z0, h0a, h0b, h0c, h0d, ss0 = _fc1_call(x, fc1_w, fc1_b.reshape(1, H),
                                            dego16)
    agg0 = _agg_kernel(h0a, h0b, h0c, h0d, edge_index)
    z1, h1a, h1b, h1c, h1d, ss1 = _layer_relu(
        z0, ss0, agg0, degi16, dego16,
        gcn_w0, gcn_b0.reshape(1, H), lin_w0, lin_b0.reshape(1, H))
    agg1 = _agg_kernel(h1a, h1b, h1c, h1d, edge_index)
    z2, h2a, h2b, h2c, h2d, ss2 = _layer_relu(
        z1, ss1, agg1, degi16, dego16,
        gcn_w1, gcn_b1.reshape(1, H), lin_w1, lin_b1.reshape(1, H))
    agg2 = _agg_kernel(h2a, h2b, h2c, h2d, edge_index)
    z3, _h3a, _h3b, _h3c, _h3d, ss3 = _layer_last(
        z2, ss2, agg2, degi16, dego16,
        gcn_w2, gcn_b2.reshape(1, H), lin_w2, lin_b2.reshape(1, H))
    value = _head_call(z3[N - 1:N], ss3, fc2_w, fc2_b.reshape(1, H),
                       sv_w, sv_b.reshape(1, 1))
    return value.reshape(1)


# speedup vs baseline: 5.3078x; 1.1162x over previous
import functools

import jax
import jax.numpy as jnp
from jax import lax
from jax.experimental import pallas as pl
from jax.experimental.pallas import tpu as pltpu, tpu_sc as plsc

N = 10000
E = 160000
FEAT = 256
H = 512
NCHUNK = 4
CW = 128
EB = 128
NBLK = E // EB
NS = 16
RPT = 624
RPT_LAST = N - 15 * RPT
BN = 400
GRID = N // BN

_mesh = plsc.VectorSubcoreMesh(core_axis_name="c", subcore_axis_name="s")


def _copy_tile_rows(sub, src_at, dst_at):
    @pl.when(sub < 15)
    def _():
        pltpu.sync_copy(src_at(RPT), dst_at(RPT))

    @pl.when(sub == 15)
    def _():
        pltpu.sync_copy(src_at(RPT_LAST), dst_at(RPT_LAST))


def _zero_vmem_2d(ref, nrows, ncols):
    ngrp = ncols // 16

    def zrow(r, _):
        def zcol(q, _):
            ref[r, pl.ds(q * 16, 16)] = jnp.zeros((16,), jnp.float32)
            return 0

        lax.fori_loop(0, ngrp, zcol, 0)
        return 0

    lax.fori_loop(0, nrows, zrow, 0)


@functools.partial(
    pl.kernel,
    mesh=_mesh,
    out_type=jax.ShapeDtypeStruct((2, N, CW), jnp.float32),
    scratch_types=[
        pltpu.VMEM((EB,), jnp.int32),
        pltpu.VMEM((EB, CW), jnp.float32),
        pltpu.VMEM((16, CW), jnp.float32),
        pltpu.VMEM_SHARED((N, CW), jnp.float32),
    ],
)
def _deg_kernel(ei_hbm, out_hbm, idx_v, ones_v, zbuf_v, acc_sh):
    core = lax.axis_index("c")
    sub = lax.axis_index("s")
    start = sub * RPT

    def fill_ones(r, _):
        def fcol(q, _):
            ones_v[r, pl.ds(q * 16, 16)] = jnp.ones((16,), jnp.float32)
            return 0

        lax.fori_loop(0, CW // 16, fcol, 0)
        return 0

    lax.fori_loop(0, EB, fill_ones, 0)
    _zero_vmem_2d(zbuf_v, 16, CW)
    nzero = (RPT // 16) + jnp.where(sub == 15, 1, 0)

    for c in range(2):
        @pl.when(core == c)
        def _(c=c):
            def zbody(k, _):
                pltpu.sync_copy(zbuf_v, acc_sh.at[pl.ds(start + k * 16, 16)])
                return 0

            lax.fori_loop(0, nzero, zbody, 0)
            plsc.subcore_barrier()
            nblk = 78 + jnp.where(sub < 2, 1, 0)

            def body(j, _):
                g = sub + NS * j
                pltpu.sync_copy(ei_hbm.at[c, pl.ds(g * EB, EB)], idx_v)
                pltpu.sync_copy(ones_v, acc_sh.at[idx_v], add=True)
                return 0

            lax.fori_loop(0, nblk, body, 0)
            plsc.subcore_barrier()
            _copy_tile_rows(sub,
                            lambda n: acc_sh.at[pl.ds(start, n)],
                            lambda n: out_hbm.at[c, pl.ds(start, n)])


@functools.partial(
    pl.kernel,
    mesh=_mesh,
    out_type=jax.ShapeDtypeStruct((NCHUNK, N, CW), jnp.float32),
    scratch_types=[
        pltpu.VMEM((EB,), jnp.int32),
        pltpu.VMEM((EB,), jnp.int32),
        pltpu.VMEM((EB,), jnp.int32),
        pltpu.VMEM((EB,), jnp.int32),
        pltpu.VMEM((EB, CW), jnp.float32),
        pltpu.VMEM((EB, CW), jnp.float32),
        pltpu.VMEM((104, CW), jnp.float32),
        pltpu.VMEM_SHARED((N, CW), jnp.float32),
        pltpu.SemaphoreType.DMA,
        pltpu.SemaphoreType.DMA,
        pltpu.SemaphoreType.DMA,
        pltpu.SemaphoreType.DMA,
    ],
)
def _agg_kernel(hn0_hbm, hn1_hbm, hn2_hbm, hn3_hbm, ei_hbm, out_hbm,
                src0_v, dst0_v, src1_v, dst1_v, rows0_v, rows1_v,
                zbuf_v, acc_sh, gsem0, gsem1, ssem0, ssem1):
    hn_chunks = (hn0_hbm, hn1_hbm, hn2_hbm, hn3_hbm)
    core = lax.axis_index("c")
    sub = lax.axis_index("s")
    start = sub * RPT
    _zero_vmem_2d(zbuf_v, 104, CW)

    for c in range(NCHUNK):
        @pl.when(core == (c % 2))
        def _(c=c):
            def zbody(k, _):
                pltpu.sync_copy(zbuf_v, acc_sh.at[pl.ds(start + k * 104, 104)])
                return 0

            lax.fori_loop(0, RPT // 104, zbody, 0)

            @pl.when(sub == 15)
            def _():
                pltpu.sync_copy(zbuf_v.at[pl.ds(0, 16)],
                                acc_sh.at[pl.ds(start + RPT, 16)])

            plsc.subcore_barrier()

            def drain(rows_v, sem):
                pltpu.make_async_copy(hn_chunks[c].at[pl.ds(0, EB)],
                                      rows_v, sem).wait()

            def pair(p, _):
                g0 = sub + NS * (2 * p)
                g1 = g0 + NS

                @pl.when(p > 0)
                def _():
                    drain(rows0_v, ssem0)

                pltpu.sync_copy(ei_hbm.at[0, pl.ds(g0 * EB, EB)], src0_v)
                pltpu.sync_copy(ei_hbm.at[1, pl.ds(g0 * EB, EB)], dst0_v)
                cp0 = pltpu.async_copy(hn_chunks[c].at[src0_v], rows0_v, gsem0)

                @pl.when(p > 0)
                def _():
                    drain(rows1_v, ssem1)

                pltpu.sync_copy(ei_hbm.at[0, pl.ds(g1 * EB, EB)], src1_v)
                pltpu.sync_copy(ei_hbm.at[1, pl.ds(g1 * EB, EB)], dst1_v)
                cp1 = pltpu.async_copy(hn_chunks[c].at[src1_v], rows1_v, gsem1)
                cp0.wait()
                pltpu.async_copy(rows0_v, acc_sh.at[dst0_v], ssem0, add=True)
                cp1.wait()
                pltpu.async_copy(rows1_v, acc_sh.at[dst1_v], ssem1, add=True)
                return 0

            lax.fori_loop(0, 39, pair, 0)
            drain(rows0_v, ssem0)
            drain(rows1_v, ssem1)

            @pl.when(sub < 2)
            def _():
                g = sub + NS * 78
                pltpu.sync_copy(ei_hbm.at[0, pl.ds(g * EB, EB)], src0_v)
                pltpu.sync_copy(ei_hbm.at[1, pl.ds(g * EB, EB)], dst0_v)
                pltpu.async_copy(hn_chunks[c].at[src0_v], rows0_v,
                                 gsem0).wait()
                pltpu.sync_copy(rows0_v, acc_sh.at[dst0_v], add=True)

            plsc.subcore_barrier()
            _copy_tile_rows(sub,
                            lambda n: acc_sh.at[pl.ds(start, n)],
                            lambda n: out_hbm.at[c, pl.ds(start, n)])


def _inv_sqrt_deg(deg16):
    return lax.rsqrt(jnp.maximum(deg16[:, 0:1], 1.0))


def _fc1_body(x_ref, w_ref, b_ref, dego_ref, z_ref, hn0_ref, hn1_ref,
              hn2_ref, hn3_ref, ss_ref):
    i = pl.program_id(0)
    z = jnp.dot(x_ref[...], w_ref[...], preferred_element_type=jnp.float32)
    z = z + b_ref[...]
    z_ref[...] = z
    hn = z * _inv_sqrt_deg(dego_ref[...])
    for c, hr in enumerate((hn0_ref, hn1_ref, hn2_ref, hn3_ref)):
        hr[...] = hn[:, c * CW:(c + 1) * CW]
    ssq = jnp.sum(z * z, axis=0, keepdims=True)

    @pl.when(i == 0)
    def _():
        ss_ref[...] = ssq

    @pl.when(i > 0)
    def _():
        ss_ref[...] = ss_ref[...] + ssq


_fc1_call = pl.pallas_call(
    _fc1_body,
    grid=(GRID,),
    in_specs=[
        pl.BlockSpec((BN, FEAT), lambda i: (i, 0)),
        pl.BlockSpec((FEAT, H), lambda i: (0, 0)),
        pl.BlockSpec((1, H), lambda i: (0, 0)),
        pl.BlockSpec((BN, CW), lambda i: (i, 0)),
    ],
    out_specs=[
        pl.BlockSpec((BN, H), lambda i: (i, 0)),
        pl.BlockSpec((BN, CW), lambda i: (i, 0)),
        pl.BlockSpec((BN, CW), lambda i: (i, 0)),
        pl.BlockSpec((BN, CW), lambda i: (i, 0)),
        pl.BlockSpec((BN, CW), lambda i: (i, 0)),
        pl.BlockSpec((1, H), lambda i: (0, 0)),
    ],
    out_shape=[
        jax.ShapeDtypeStruct((N, H), jnp.float32),
        jax.ShapeDtypeStruct((N, CW), jnp.float32),
        jax.ShapeDtypeStruct((N, CW), jnp.float32),
        jax.ShapeDtypeStruct((N, CW), jnp.float32),
        jax.ShapeDtypeStruct((N, CW), jnp.float32),
        jax.ShapeDtypeStruct((1, H), jnp.float32),
    ],
)


def _layer_body(z_ref, ssp_ref, agg_ref, degi_ref, dego_ref,
                wg_ref, bg_ref, wl_ref, bl_ref,
                zo_ref, hn0_ref, hn1_ref, hn2_ref, hn3_ref, ss_ref, *, relu):
    i = pl.program_id(0)
    inv_cn = 1.0 / jnp.maximum(jnp.sqrt(ssp_ref[...]), 1e-12)
    h = z_ref[...] * inv_cn
    agg = jnp.concatenate(
        [agg_ref[0], agg_ref[1], agg_ref[2], agg_ref[3]], axis=1)
    aggs = agg * inv_cn * _inv_sqrt_deg(degi_ref[...])
    pre = jnp.dot(aggs, wg_ref[...], preferred_element_type=jnp.float32)
    pre = pre + bg_ref[...]
    pre = pre + jnp.dot(h, wl_ref[...], preferred_element_type=jnp.float32)
    pre = pre + bl_ref[...]
    z = jnp.maximum(pre, 0.0) if relu else pre
    zo_ref[...] = z
    hn = z * _inv_sqrt_deg(dego_ref[...])
    for c, hr in enumerate((hn0_ref, hn1_ref, hn2_ref, hn3_ref)):
        hr[...] = hn[:, c * CW:(c + 1) * CW]
    ssq = jnp.sum(z * z, axis=0, keepdims=True)

    @pl.when(i == 0)
    def _():
        ss_ref[...] = ssq

    @pl.when(i > 0)
    def _():
        ss_ref[...] = ss_ref[...] + ssq


def _make_layer_call(relu):
    return pl.pallas_call(
        functools.partial(_layer_body, relu=relu),
        grid=(GRID,),
        in_specs=[
            pl.BlockSpec((BN, H), lambda i: (i, 0)),
            pl.BlockSpec((1, H), lambda i: (0, 0)),
            pl.BlockSpec((NCHUNK, BN, CW), lambda i: (0, i, 0)),
            pl.BlockSpec((BN, CW), lambda i: (i, 0)),
            pl.BlockSpec((BN, CW), lambda i: (i, 0)),
            pl.BlockSpec((H, H), lambda i: (0, 0)),
            pl.BlockSpec((1, H), lambda i: (0, 0)),
            pl.BlockSpec((H, H), lambda i: (0, 0)),
            pl.BlockSpec((1, H), lambda i: (0, 0)),
        ],
        out_specs=[
            pl.BlockSpec((BN, H), lambda i: (i, 0)),
            pl.BlockSpec((BN, CW), lambda i: (i, 0)),
            pl.BlockSpec((BN, CW), lambda i: (i, 0)),
            pl.BlockSpec((BN, CW), lambda i: (i, 0)),
            pl.BlockSpec((BN, CW), lambda i: (i, 0)),
            pl.BlockSpec((1, H), lambda i: (0, 0)),
        ],
        out_shape=[
            jax.ShapeDtypeStruct((N, H), jnp.float32),
            jax.ShapeDtypeStruct((N, CW), jnp.float32),
            jax.ShapeDtypeStruct((N, CW), jnp.float32),
            jax.ShapeDtypeStruct((N, CW), jnp.float32),
            jax.ShapeDtypeStruct((N, CW), jnp.float32),
            jax.ShapeDtypeStruct((1, H), jnp.float32),
        ],
    )


_layer_relu = _make_layer_call(True)
_layer_last = _make_layer_call(False)


def _head_body(row_ref, ss_ref, w2_ref, b2_ref, sv_ref, svb_ref, out_ref):
    r = row_ref[...] / jnp.maximum(jnp.sqrt(ss_ref[...]), 1e-12)
    h3 = jnp.dot(r, w2_ref[...], preferred_element_type=jnp.float32)
    h3 = jnp.maximum(h3 + b2_ref[...], 0.0)
    out_ref[...] = jnp.dot(h3, sv_ref[...],
                           preferred_element_type=jnp.float32) + svb_ref[...]


_head_call = pl.pallas_call(
    _head_body,
    out_shape=jax.ShapeDtypeStruct((1, 1), jnp.float32),
)


def kernel(x, edge_index, fc1_w, fc1_b, gcn_w0, gcn_b0, gcn_w1, gcn_b1,
           gcn_w2, gcn_b2, lin_w0, lin_b0, lin_w1, lin_b1, lin_w2, lin_b2,
           fc2_w, fc2_b, sv_w, sv_b):
    deg16 = _deg_kernel(edge_index)
    dego16 = deg16[0]
    degi16 = deg16[1]

    z0, h0a, h0b, h0c, h0d, ss0 = _fc1_call(x, fc1_w, fc1_b.reshape(1, H),
                                            dego16)
    agg0 = _agg_kernel(h0a, h0b, h0c, h0d, edge_index)
    z1, h1a, h1b, h1c, h1d, ss1 = _layer_relu(
        z0, ss0, agg0, degi16, dego16,
        gcn_w0, gcn_b0.reshape(1, H), lin_w0, lin_b0.reshape(1, H))
    agg1 = _agg_kernel(h1a, h1b, h1c, h1d, edge_index)
    z2, h2a, h2b, h2c, h2d, ss2 = _layer_relu(
        z1, ss1, agg1, degi16, dego16,
        gcn_w1, gcn_b1.reshape(1, H), lin_w1, lin_b1.reshape(1, H))
    agg2 = _agg_kernel(h2a, h2b, h2c, h2d, edge_index)
    z3, _h3a, _h3b, _h3c, _h3d, ss3 = _layer_last(
        z2, ss2, agg2, degi16, dego16,
        gcn_w2, gcn_b2.reshape(1, H), lin_w2, lin_b2.reshape(1, H))
    value = _head_call(z3[N - 1:N], ss3, fc2_w, fc2_b.reshape(1, H),
                       sv_w, sv_b.reshape(1, 1))
    return value.reshape(1)
